# baseline (speedup 1.0000x reference)
import jax
import jax.numpy as jnp
from jax.experimental import pallas as pl


def _body(inp_ref, tab_ref, out_ref):
    out_ref[...] = inp_ref[...] + tab_ref[...][None, :, :]


def kernel(inputs, table):
    B = inputs.shape[0]
    P, D = table.shape
    B_BLK = 32
    out = pl.pallas_call(
        _body,
        grid=(B // B_BLK,),
        in_specs=[
            pl.BlockSpec((B_BLK, 1, D), lambda i: (i, 0, 0)),
            pl.BlockSpec((P, D), lambda i: (0, 0)),
        ],
        out_specs=pl.BlockSpec((B_BLK, P, D), lambda i: (i, 0, 0)),
        out_shape=jax.ShapeDtypeStruct((B, P, D), jnp.float32),
    )(inputs, table)
    return out


# speedup vs baseline: 6.2419x; 6.2419x over previous
import jax
import jax.numpy as jnp
from jax.experimental import pallas as pl


def _body(inp_ref, tab_ref, out_ref):
    tab = tab_ref[...][:, :, None]
    inp = inp_ref[...][None, :, :]
    out_ref[...] = tab + inp


def kernel(inputs, table):
    B = inputs.shape[0]
    P, D = table.shape
    inp_t = inputs.reshape(B, D).T
    P_BLK = 24
    n_blk = -(-P // P_BLK)
    tab_pad = jnp.pad(table, ((0, n_blk * P_BLK - P), (0, 0)))
    phys = pl.pallas_call(
        _body,
        grid=(n_blk,),
        in_specs=[
            pl.BlockSpec((D, B), lambda i: (0, 0)),
            pl.BlockSpec((P_BLK, D), lambda i: (i, 0)),
        ],
        out_specs=pl.BlockSpec((P_BLK, D, B), lambda i: (i, 0, 0)),
        out_shape=jax.ShapeDtypeStruct((P, D, B), jnp.float32),
    )(inp_t, tab_pad)
    return phys.transpose(2, 0, 1)
